# Initial kernel scaffold; baseline (speedup 1.0000x reference)
import jax
import jax.numpy as jnp
from jax.experimental import pallas as pl


def kernel(inputs):
    raise NotImplementedError("write your pallas kernel here")


# speedup vs baseline: 110.1102x; 110.1102x over previous
import functools

import jax
import jax.numpy as jnp
from jax import lax
from jax.experimental import pallas as pl
from jax.experimental.pallas import tpu as pltpu
from jax.experimental.pallas import tpu_sc as plsc

ROWS = 2048
COLS = 8192
TOTAL = ROWS * COLS
NUM_CORES = 2
NUM_SUBCORES = 16
NW = NUM_CORES * NUM_SUBCORES
PER_W = TOTAL // NW
CHUNK = 16384
N_CHUNKS = PER_W // CHUNK
LANES = 16

_GDN = lax.GatherDimensionNumbers(
    offset_dims=(), collapsed_slice_dims=(0,), start_index_map=(0,))


def _vgather(v, idx):
    return lax.gather(v, idx[:, None], _GDN, slice_sizes=(1,),
                      mode=lax.GatherScatterMode.PROMISE_IN_BOUNDS)


_mesh = plsc.VectorSubcoreMesh(core_axis_name="c", subcore_axis_name="s")


@functools.partial(
    pl.kernel,
    mesh=_mesh,
    out_type=jax.ShapeDtypeStruct((TOTAL,), jnp.float32),
    scratch_types=[
        pltpu.VMEM((CHUNK,), jnp.float32),
        pltpu.VMEM((CHUNK,), jnp.float32),
    ],
)
def _sc_prune(x_hbm, out_hbm, in_v, out_v):
    wid = lax.axis_index("s") * NUM_CORES + lax.axis_index("c")
    base = wid * PER_W

    lane = lax.iota(jnp.int32, LANES)
    p1 = lane ^ 1
    p2 = lane ^ 2
    p3 = lane ^ 3
    t1 = p1 < lane
    t2 = p2 < lane
    t3 = p3 < lane

    def vreg_body(it, carry):
        x = in_v[pl.ds(it * LANES, LANES)]
        a = jnp.abs(x)
        b1 = _vgather(a, p1)
        b2 = _vgather(a, p2)
        b3 = _vgather(a, p3)
        c1 = (b1 < a) | ((b1 == a) & t1)
        c2 = (b2 < a) | ((b2 == a) & t2)
        c3 = (b3 < a) | ((b3 == a) & t3)
        keep = (c1 & c2) | (c1 & c3) | (c2 & c3)
        out_v[pl.ds(it * LANES, LANES)] = jnp.where(keep, x, 0.0)
        return carry

    def chunk_body(ci, carry):
        off = base + ci * CHUNK
        pltpu.sync_copy(x_hbm.at[pl.ds(off, CHUNK)], in_v)
        lax.fori_loop(0, CHUNK // LANES, vreg_body, 0, unroll=4)
        pltpu.sync_copy(out_v, out_hbm.at[pl.ds(off, CHUNK)])
        return carry

    lax.fori_loop(0, N_CHUNKS, chunk_body, 0)


def kernel(inputs):
    out = _sc_prune(inputs.reshape(-1))
    return out.reshape(inputs.shape)
